# baseline (speedup 1.0000x reference)
import jax
import jax.numpy as jnp
from jax.experimental import pallas as pl
from jax.experimental.pallas import tpu as pltpu

F32 = jnp.float32
NSEL = 300
B, N, D, L = 8, 20000, 256, 92
BN = 2000
NBLK = N // BN
RPAD = 157
CAP = 384
HI = jax.lax.Precision.HIGHEST


def _iotaf(shape, dim):
    return jax.lax.broadcasted_iota(jnp.int32, shape, dim).astype(F32)


def _refine_threshold(vals, target):
    lo = jnp.float32(0.0)
    step = jnp.float32(8.0)
    for _ in range(3):
        thr = lo + _iotaf((1, 32, 1), 1) * step
        ge = (vals[:, None, :] >= thr).astype(F32)
        cnt = jnp.sum(ge, axis=(0, 2), keepdims=True)
        lo = jnp.max(jnp.where(cnt >= target, thr, lo))
        step = step * jnp.float32(1.0 / 32.0)
    return lo


def _positions(maskf, tri_incl, tri_strict):
    cs = jnp.dot(maskf, tri_incl, precision=HI, preferred_element_type=F32)
    rowtot = cs[:, -1:]
    off = jnp.dot(tri_strict, rowtot, precision=HI, preferred_element_type=F32)
    return off + cs - maskf


def _tril(n):
    a = _iotaf((n, n), 0)
    b_ = _iotaf((n, n), 1)
    return (a > b_).astype(F32)


def _tri(n, strict):
    a = _iotaf((n, n), 0)
    b_ = _iotaf((n, n), 1)
    return ((a < b_) if strict else (a <= b_)).astype(F32)


def _col_to_row(c, ident):
    return jax.lax.dot_general(c, ident, (((0,), (0,)), ((), ())),
                               precision=HI, preferred_element_type=F32)


def _k1(logits_ref, wt_ref, rmax_ref):
    p = jax.nn.sigmoid(logits_ref[0])
    pr = jnp.dot(p, wt_ref[...], preferred_element_type=F32)
    rmax_ref[0, 0] = jnp.max(pr, axis=1, keepdims=True)


def _run_k1(pred_logits, wt):
    return pl.pallas_call(
        _k1,
        grid=(B, NBLK),
        compiler_params=pltpu.CompilerParams(
            dimension_semantics=("parallel", "parallel")),
        in_specs=[
            pl.BlockSpec((1, BN, D), lambda b, n: (b, n, 0)),
            pl.BlockSpec((D, L), lambda b, n: (0, 0)),
        ],
        out_specs=pl.BlockSpec((1, 1, BN, 1), lambda b, n: (b, n, 0, 0)),
        out_shape=jax.ShapeDtypeStruct((B, NBLK, BN, 1), F32),
    )(pred_logits, wt)


def _k2(rm_ref, rows_ref):
    rm = rm_ref[0]
    t = _refine_threshold(rm, jnp.float32(NSEL))
    mask = rm >= t
    mf = mask.astype(F32)
    cs = jnp.dot(mf, _tri(128, False), precision=HI,
                 preferred_element_type=F32)
    rowtot = cs[:, -1:]
    off = jnp.dot(_tril(RPAD), rowtot, precision=HI,
                  preferred_element_type=F32)
    inclr = off + rowtot
    c_tot = jnp.sum(mf)
    identr = (jax.lax.broadcasted_iota(jnp.int32, (RPAD, RPAD), 0) ==
              jax.lax.broadcasted_iota(jnp.int32, (RPAD, RPAD), 1)).astype(F32)
    inclr_row = _col_to_row(inclr, identr)
    off_row = _col_to_row(off, identr)
    pcol = _iotaf((CAP, 1), 0)
    rp = jnp.sum((inclr_row <= pcol).astype(F32), axis=1, keepdims=True)
    rio = _iotaf((1, RPAD), 1)
    oh2 = (rp == rio).astype(F32)
    startp = jnp.sum(oh2 * off_row, axis=1, keepdims=True)
    q = pcol - startp
    csm = jnp.where(mask, cs - mf, -1.0)
    csg = jnp.dot(oh2, csm, precision=HI, preferred_element_type=F32)
    ohc = csg == q
    laneio = _iotaf((CAP, 128), 1)
    lanesel = jnp.sum(jnp.where(ohc, laneio, 0.0), axis=1, keepdims=True)
    rowid = rp * 128.0 + lanesel
    rows_ref[0] = jnp.where(pcol < c_tot, rowid + 1.0, 0.0)


def _run_k2(rm):
    return pl.pallas_call(
        _k2,
        grid=(B,),
        compiler_params=pltpu.CompilerParams(
            dimension_semantics=("parallel",)),
        in_specs=[pl.BlockSpec((1, RPAD, 128), lambda b: (b, 0, 0))],
        out_specs=pl.BlockSpec((1, CAP, 1), lambda b: (b, 0, 0)),
        out_shape=jax.ShapeDtypeStruct((B, CAP, 1), F32),
    )(rm)


def _k3(logits_ref, wt_ref, rowsf_ref, rowsi_ref, sc_ref, lab_ref, br_ref,
        lrows, sem):
    b = pl.program_id(0)

    def start_body(i, carry):
        r = rowsi_ref[b, i]
        pltpu.make_async_copy(logits_ref.at[b, pl.ds(r, 1), :],
                              lrows.at[pl.ds(i, 1), :], sem).start()
        return carry

    jax.lax.fori_loop(0, CAP, start_body, 0)

    def wait_body(i, carry):
        r = rowsi_ref[b, i]
        pltpu.make_async_copy(logits_ref.at[b, pl.ds(r, 1), :],
                              lrows.at[pl.ds(i, 1), :], sem).wait()
        return carry

    jax.lax.fori_loop(0, CAP, wait_body, 0)
    cand = jnp.dot(jax.nn.sigmoid(lrows[...]), wt_ref[...],
                   preferred_element_type=F32)
    rowsf = rowsf_ref[0]
    valid = rowsf >= 0.0
    V = jnp.where(valid, cand, -1.0)
    te = _refine_threshold(V, jnp.float32(NSEL))
    me = V >= te
    mf = me.astype(F32)
    cs = jnp.dot(mf, _tri(L, False), precision=HI,
                 preferred_element_type=F32)
    rowtot = cs[:, -1:]
    off = jnp.dot(_tril(CAP), rowtot, precision=HI,
                  preferred_element_type=F32)
    inclr = off + rowtot
    ident = (jax.lax.broadcasted_iota(jnp.int32, (CAP, CAP), 0) ==
             jax.lax.broadcasted_iota(jnp.int32, (CAP, CAP), 1)).astype(F32)
    inclr_row = _col_to_row(inclr, ident)
    off_row = _col_to_row(off, ident)
    rowsf_row = _col_to_row(rowsf, ident)
    pcol = _iotaf((CAP, 1), 0)
    rp = jnp.sum((inclr_row <= pcol).astype(F32), axis=1, keepdims=True)
    rio = _iotaf((1, CAP), 1)
    oh2 = (rp == rio).astype(F32)
    startp = jnp.sum(oh2 * off_row, axis=1, keepdims=True)
    q = pcol - startp
    csm = jnp.where(me, cs - mf, -1.0)
    csg = jnp.dot(oh2, csm, precision=HI, preferred_element_type=F32)
    vg = jnp.dot(oh2, V, precision=HI, preferred_element_type=F32)
    ohc = csg == q
    colio = _iotaf((CAP, L), 1)
    sv = jnp.sum(jnp.where(ohc, vg, 0.0), axis=1, keepdims=True)
    scol = jnp.sum(jnp.where(ohc, colio, 0.0), axis=1, keepdims=True)
    srow = jnp.sum(oh2 * rowsf_row, axis=1, keepdims=True)
    ce = jnp.sum(mf)
    sl = _iotaf((CAP, 1), 0) < ce
    vv = jnp.where(sl, sv, -1.0)
    vflat = jnp.where(sl, srow * jnp.float32(L) + scol, 4.0e6)
    ident = (jax.lax.broadcasted_iota(jnp.int32, (CAP, CAP), 0) ==
             jax.lax.broadcasted_iota(jnp.int32, (CAP, CAP), 1)).astype(F32)
    vvr = _col_to_row(vv, ident)
    vfr = _col_to_row(vflat, ident)
    gt = (vvr > vv) | ((vvr == vv) & (vfr < vflat))
    rank = jnp.sum(gt.astype(F32), axis=1, keepdims=True)
    rio = _iotaf((1, CAP), 1)
    oh_sel = (rank == rio).astype(F32)
    sc_ref[0] = jnp.sum(oh_sel * vv, axis=0, keepdims=True)
    lab_ref[0] = jnp.sum(oh_sel * scol, axis=0, keepdims=True)
    br_ref[0] = jnp.sum(oh_sel * srow, axis=0, keepdims=True)


def _run_k3(pred_logits, wt, rowsf, rowsi):
    out = pl.BlockSpec((1, 1, CAP), lambda b: (b, 0, 0))
    return pl.pallas_call(
        _k3,
        grid=(B,),
        compiler_params=pltpu.CompilerParams(
            dimension_semantics=("parallel",)),
        in_specs=[
            pl.BlockSpec(memory_space=pl.ANY),
            pl.BlockSpec((D, L), lambda b: (0, 0)),
            pl.BlockSpec((1, CAP, 1), lambda b: (b, 0, 0)),
            pl.BlockSpec(memory_space=pltpu.SMEM),
        ],
        out_specs=[out, out, out],
        out_shape=[jax.ShapeDtypeStruct((B, 1, CAP), F32)] * 3,
        scratch_shapes=[pltpu.VMEM((CAP, D), F32),
                        pltpu.SemaphoreType.DMA],
    )(pred_logits, wt, rowsf, rowsi)


def _k5(br_ref, pb_ref, tsz_ref, out_ref):
    b = pl.program_id(0)
    br = br_ref[0]
    CH = 2048
    acc = jnp.zeros((CAP, 4), F32)
    for c0 in range(0, N, CH):
        cw = min(CH, N - c0)
        lio = _iotaf((1, cw), 1) + jnp.float32(c0)
        ohc = (br == lio).astype(F32)
        acc = acc + jnp.dot(ohc, pb_ref[0, c0:c0 + cw, :],
                            precision=HI, preferred_element_type=F32)
    cx, cy, w, h = acc[:, 0:1], acc[:, 1:2], acc[:, 2:3], acc[:, 3:4]
    x0 = cx - 0.5 * w
    y0 = cy - 0.5 * h
    x1 = cx + 0.5 * w
    y1 = cy + 0.5 * h
    wf = tsz_ref[b, 1].astype(F32)
    hf = tsz_ref[b, 0].astype(F32)
    out_ref[0] = jnp.concatenate([x0 * wf, y0 * hf, x1 * wf, y1 * hf], axis=1)


def _run_k5(browf, pred_boxes, target_sizes):
    return pl.pallas_call(
        _k5,
        grid=(B,),
        compiler_params=pltpu.CompilerParams(
            dimension_semantics=("parallel",)),
        in_specs=[
            pl.BlockSpec((1, CAP, 1), lambda b: (b, 0, 0)),
            pl.BlockSpec((1, N, 4), lambda b: (b, 0, 0)),
            pl.BlockSpec(memory_space=pltpu.SMEM),
        ],
        out_specs=pl.BlockSpec((1, CAP, 4), lambda b: (b, 0, 0)),
        out_shape=jax.ShapeDtypeStruct((B, CAP, 4), F32),
    )(browf, pred_boxes, target_sizes)


@jax.jit
def kernel(pred_logits, pred_boxes, target_sizes, positive_maps):
    wt = jnp.swapaxes(positive_maps, 1, 2)[0]
    rowmax4 = _run_k1(pred_logits, wt)
    rowmax = rowmax4.reshape(B, N)
    rm = jnp.pad(rowmax, ((0, 0), (0, RPAD * 128 - N)),
                 constant_values=-1.0).reshape(B, RPAD, 128)
    rows_raw = _run_k2(rm)
    rowsf = rows_raw - 1.0
    rowsi = jnp.maximum(rows_raw[:, :, 0].astype(jnp.int32) - 1, 0)
    scores, labf, browf = _run_k3(pred_logits, wt, rowsf, rowsi)
    boxes = _run_k5(browf.reshape(B, CAP, 1), pred_boxes, target_sizes)
    return (scores[:, 0, :NSEL],
            labf[:, 0, :NSEL].astype(jnp.int32),
            boxes[:, :NSEL, :])


# speedup vs baseline: 1.4961x; 1.4961x over previous
import jax
import jax.numpy as jnp
from jax.experimental import pallas as pl
from jax.experimental.pallas import tpu as pltpu

F32 = jnp.float32
NSEL = 300
B, N, D, L = 8, 20000, 256, 92
BN = 2000
NBLK = N // BN
RPAD = 157
CAP = 384
HI = jax.lax.Precision.HIGHEST


def _iotaf(shape, dim):
    return jax.lax.broadcasted_iota(jnp.int32, shape, dim).astype(F32)


def _refine_threshold(vals, target):
    lo = jnp.float32(0.0)
    step = jnp.float32(8.0)
    for _ in range(3):
        thr = lo + _iotaf((1, 32, 1), 1) * step
        ge = (vals[:, None, :] >= thr).astype(F32)
        cnt = jnp.sum(ge, axis=(0, 2), keepdims=True)
        lo = jnp.max(jnp.where(cnt >= target, thr, lo))
        step = step * jnp.float32(1.0 / 32.0)
    return lo


def _positions(maskf, tri_incl, tri_strict):
    cs = jnp.dot(maskf, tri_incl, precision=HI, preferred_element_type=F32)
    rowtot = cs[:, -1:]
    off = jnp.dot(tri_strict, rowtot, precision=HI, preferred_element_type=F32)
    return off + cs - maskf


def _tril(n):
    a = _iotaf((n, n), 0)
    b_ = _iotaf((n, n), 1)
    return (a > b_).astype(F32)


def _tri(n, strict):
    a = _iotaf((n, n), 0)
    b_ = _iotaf((n, n), 1)
    return ((a < b_) if strict else (a <= b_)).astype(F32)


def _col_to_row(c, ident):
    return jax.lax.dot_general(c, ident, (((0,), (0,)), ((), ())),
                               precision=HI, preferred_element_type=F32)


def _k1(logits_ref, wt_ref, rmax_ref):
    p = jax.nn.sigmoid(logits_ref[0])
    pr = jnp.dot(p, wt_ref[...], preferred_element_type=F32)
    rmax_ref[0, 0] = jnp.max(pr, axis=1, keepdims=True)


def _run_k1(pred_logits, wt):
    return pl.pallas_call(
        _k1,
        grid=(B, NBLK),
        compiler_params=pltpu.CompilerParams(
            dimension_semantics=("parallel", "parallel")),
        in_specs=[
            pl.BlockSpec((1, BN, D), lambda b, n: (b, n, 0)),
            pl.BlockSpec((D, L), lambda b, n: (0, 0)),
        ],
        out_specs=pl.BlockSpec((1, 1, BN, 1), lambda b, n: (b, n, 0, 0)),
        out_shape=jax.ShapeDtypeStruct((B, NBLK, BN, 1), F32),
    )(pred_logits, wt)


def _k2(rm_ref, rows_ref):
    rm = rm_ref[0]
    t = _refine_threshold(rm, jnp.float32(NSEL))
    mask = rm >= t
    mf = mask.astype(F32)
    cs = jnp.dot(mf, _tri(128, False), precision=HI,
                 preferred_element_type=F32)
    rowtot = cs[:, -1:]
    off = jnp.dot(_tril(RPAD), rowtot, precision=HI,
                  preferred_element_type=F32)
    inclr = off + rowtot
    c_tot = jnp.sum(mf)
    identr = (jax.lax.broadcasted_iota(jnp.int32, (RPAD, RPAD), 0) ==
              jax.lax.broadcasted_iota(jnp.int32, (RPAD, RPAD), 1)).astype(F32)
    inclr_row = _col_to_row(inclr, identr)
    off_row = _col_to_row(off, identr)
    pcol = _iotaf((CAP, 1), 0)
    rp = jnp.sum((inclr_row <= pcol).astype(F32), axis=1, keepdims=True)
    rio = _iotaf((1, RPAD), 1)
    oh2 = (rp == rio).astype(F32)
    startp = jnp.sum(oh2 * off_row, axis=1, keepdims=True)
    q = pcol - startp
    csm = jnp.where(mask, cs - mf, -1.0)
    csg = jnp.dot(oh2, csm, precision=HI, preferred_element_type=F32)
    ohc = csg == q
    laneio = _iotaf((CAP, 128), 1)
    lanesel = jnp.sum(jnp.where(ohc, laneio, 0.0), axis=1, keepdims=True)
    rowid = rp * 128.0 + lanesel
    rows_ref[0] = jnp.where(pcol < c_tot, rowid + 1.0, 0.0)


def _run_k2(rm):
    return pl.pallas_call(
        _k2,
        grid=(B,),
        compiler_params=pltpu.CompilerParams(
            dimension_semantics=("parallel",)),
        in_specs=[pl.BlockSpec((1, RPAD, 128), lambda b: (b, 0, 0))],
        out_specs=pl.BlockSpec((1, CAP, 1), lambda b: (b, 0, 0)),
        out_shape=jax.ShapeDtypeStruct((B, CAP, 1), F32),
    )(rm)


def _k3(logits_ref, wt_ref, rowsf_ref, rowsi_ref, sc_ref, lab_ref, br_ref,
        lrows, sem):
    b = pl.program_id(0)

    def start_body(i, carry):
        r = rowsi_ref[b, i]
        pltpu.make_async_copy(logits_ref.at[b, pl.ds(r, 1), :],
                              lrows.at[pl.ds(i, 1), :], sem).start()
        return carry

    jax.lax.fori_loop(0, CAP, start_body, 0)

    def wait_body(i, carry):
        r = rowsi_ref[b, i]
        pltpu.make_async_copy(logits_ref.at[b, pl.ds(r, 1), :],
                              lrows.at[pl.ds(i, 1), :], sem).wait()
        return carry

    jax.lax.fori_loop(0, CAP, wait_body, 0)
    cand = jnp.dot(jax.nn.sigmoid(lrows[...]), wt_ref[...],
                   preferred_element_type=F32)
    rowsf = rowsf_ref[0]
    valid = rowsf >= 0.0
    V = jnp.where(valid, cand, -1.0)
    te = _refine_threshold(V, jnp.float32(NSEL))
    me = V >= te
    mf = me.astype(F32)
    cs = jnp.dot(mf, _tri(L, False), precision=HI,
                 preferred_element_type=F32)
    rowtot = cs[:, -1:]
    off = jnp.dot(_tril(CAP), rowtot, precision=HI,
                  preferred_element_type=F32)
    inclr = off + rowtot
    ident = (jax.lax.broadcasted_iota(jnp.int32, (CAP, CAP), 0) ==
             jax.lax.broadcasted_iota(jnp.int32, (CAP, CAP), 1)).astype(F32)
    inclr_row = _col_to_row(inclr, ident)
    off_row = _col_to_row(off, ident)
    rowsf_row = _col_to_row(rowsf, ident)
    pcol = _iotaf((CAP, 1), 0)
    rp = jnp.sum((inclr_row <= pcol).astype(F32), axis=1, keepdims=True)
    rio = _iotaf((1, CAP), 1)
    oh2 = (rp == rio).astype(F32)
    startp = jnp.sum(oh2 * off_row, axis=1, keepdims=True)
    q = pcol - startp
    csm = jnp.where(me, cs - mf, -1.0)
    csg = jnp.dot(oh2, csm, precision=HI, preferred_element_type=F32)
    vg = jnp.dot(oh2, V, precision=HI, preferred_element_type=F32)
    ohc = csg == q
    colio = _iotaf((CAP, L), 1)
    sv = jnp.sum(jnp.where(ohc, vg, 0.0), axis=1, keepdims=True)
    scol = jnp.sum(jnp.where(ohc, colio, 0.0), axis=1, keepdims=True)
    srow = jnp.sum(oh2 * rowsf_row, axis=1, keepdims=True)
    ce = jnp.sum(mf)
    sl = _iotaf((CAP, 1), 0) < ce
    vv = jnp.where(sl, sv, -1.0)
    vflat = jnp.where(sl, srow * jnp.float32(L) + scol, 4.0e6)
    ident = (jax.lax.broadcasted_iota(jnp.int32, (CAP, CAP), 0) ==
             jax.lax.broadcasted_iota(jnp.int32, (CAP, CAP), 1)).astype(F32)
    vvr = _col_to_row(vv, ident)
    vfr = _col_to_row(vflat, ident)
    gt = (vvr > vv) | ((vvr == vv) & (vfr < vflat))
    rank = jnp.sum(gt.astype(F32), axis=1, keepdims=True)
    rio = _iotaf((1, CAP), 1)
    oh_sel = (rank == rio).astype(F32)
    sc_ref[0] = jnp.sum(oh_sel * vv, axis=0, keepdims=True)
    lab_ref[0] = jnp.sum(oh_sel * scol, axis=0, keepdims=True)
    br_ref[0] = jnp.sum(oh_sel * srow, axis=0, keepdims=True)


def _run_k3(pred_logits, wt, rowsf, rowsi):
    out = pl.BlockSpec((1, 1, CAP), lambda b: (b, 0, 0))
    return pl.pallas_call(
        _k3,
        grid=(B,),
        compiler_params=pltpu.CompilerParams(
            dimension_semantics=("parallel",)),
        in_specs=[
            pl.BlockSpec(memory_space=pl.ANY),
            pl.BlockSpec((D, L), lambda b: (0, 0)),
            pl.BlockSpec((1, CAP, 1), lambda b: (b, 0, 0)),
            pl.BlockSpec(memory_space=pltpu.SMEM),
        ],
        out_specs=[out, out, out],
        out_shape=[jax.ShapeDtypeStruct((B, 1, CAP), F32)] * 3,
        scratch_shapes=[pltpu.VMEM((CAP, D), F32),
                        pltpu.SemaphoreType.DMA],
    )(pred_logits, wt, rowsf, rowsi)


def _k5(browi_ref, pb_ref, tsz_ref, out_ref):
    b = pl.program_id(0)

    def body(i, carry):
        r = browi_ref[b, i]
        out_ref[0, pl.ds(i, 1), :] = pb_ref[0, pl.ds(r, 1), :]
        return carry

    jax.lax.fori_loop(0, CAP, body, 0)
    g = out_ref[0]
    cx, cy, w, h = g[:, 0:1], g[:, 1:2], g[:, 2:3], g[:, 3:4]
    x0 = cx - 0.5 * w
    y0 = cy - 0.5 * h
    x1 = cx + 0.5 * w
    y1 = cy + 0.5 * h
    wf = tsz_ref[b, 1].astype(F32)
    hf = tsz_ref[b, 0].astype(F32)
    out_ref[0] = jnp.concatenate([x0 * wf, y0 * hf, x1 * wf, y1 * hf], axis=1)


def _run_k5(browi, pred_boxes, target_sizes):
    return pl.pallas_call(
        _k5,
        grid=(B,),
        compiler_params=pltpu.CompilerParams(
            dimension_semantics=("parallel",)),
        in_specs=[
            pl.BlockSpec(memory_space=pltpu.SMEM),
            pl.BlockSpec((1, N, 4), lambda b: (b, 0, 0)),
            pl.BlockSpec(memory_space=pltpu.SMEM),
        ],
        out_specs=pl.BlockSpec((1, CAP, 4), lambda b: (b, 0, 0)),
        out_shape=jax.ShapeDtypeStruct((B, CAP, 4), F32),
    )(browi, pred_boxes, target_sizes)


@jax.jit
def kernel(pred_logits, pred_boxes, target_sizes, positive_maps):
    wt = jnp.swapaxes(positive_maps, 1, 2)[0]
    rowmax4 = _run_k1(pred_logits, wt)
    rowmax = rowmax4.reshape(B, N)
    rm = jnp.pad(rowmax, ((0, 0), (0, RPAD * 128 - N)),
                 constant_values=-1.0).reshape(B, RPAD, 128)
    rows_raw = _run_k2(rm)
    rowsf = rows_raw - 1.0
    rowsi = jnp.maximum(rows_raw[:, :, 0].astype(jnp.int32) - 1, 0)
    scores, labf, browf = _run_k3(pred_logits, wt, rowsf, rowsi)
    browi = jnp.clip(browf[:, 0, :].astype(jnp.int32), 0, N - 1)
    boxes = _run_k5(browi, pred_boxes, target_sizes)
    return (scores[:, 0, :NSEL],
            labf[:, 0, :NSEL].astype(jnp.int32),
            boxes[:, :NSEL, :])


# speedup vs baseline: 1.4981x; 1.0014x over previous
import jax
import jax.numpy as jnp
from jax.experimental import pallas as pl
from jax.experimental.pallas import tpu as pltpu

F32 = jnp.float32
NSEL = 300
B, N, D, L = 8, 20000, 256, 92
BN = 2000
NBLK = N // BN
RPAD = 157
CAP = 384
HI = jax.lax.Precision.HIGHEST


def _iotaf(shape, dim):
    return jax.lax.broadcasted_iota(jnp.int32, shape, dim).astype(F32)


def _refine_threshold(vals, target):
    lo = jnp.float32(0.0)
    step = jnp.float32(8.0)
    for _ in range(3):
        thr = lo + _iotaf((1, 32, 1), 1) * step
        ge = (vals[:, None, :] >= thr).astype(F32)
        cnt = jnp.sum(ge, axis=(0, 2), keepdims=True)
        lo = jnp.max(jnp.where(cnt >= target, thr, lo))
        step = step * jnp.float32(1.0 / 32.0)
    return lo


def _tril(n):
    a = _iotaf((n, n), 0)
    b_ = _iotaf((n, n), 1)
    return (a > b_).astype(F32)


def _tri_incl(n):
    a = _iotaf((n, n), 0)
    b_ = _iotaf((n, n), 1)
    return (a <= b_).astype(F32)


def _col_to_row(c, ident):
    return jax.lax.dot_general(c, ident, (((0,), (0,)), ((), ())),
                               precision=HI, preferred_element_type=F32)


def _k1(logits_ref, wt_ref, rmax_ref):
    p = jax.nn.sigmoid(logits_ref[0])
    pr = jnp.dot(p, wt_ref[...], preferred_element_type=F32)
    rmax_ref[0, 0] = jnp.max(pr, axis=1, keepdims=True)


def _run_k1(pred_logits, wt):
    return pl.pallas_call(
        _k1,
        grid=(B, NBLK),
        compiler_params=pltpu.CompilerParams(
            dimension_semantics=("parallel", "parallel")),
        in_specs=[
            pl.BlockSpec((1, BN, D), lambda b, n: (b, n, 0)),
            pl.BlockSpec((D, L), lambda b, n: (0, 0)),
        ],
        out_specs=pl.BlockSpec((1, 1, BN, 1), lambda b, n: (b, n, 0, 0)),
        out_shape=jax.ShapeDtypeStruct((B, NBLK, BN, 1), F32),
    )(pred_logits, wt)


def _k2(rm_ref, rows_ref):
    rm = rm_ref[0]
    t = _refine_threshold(rm, jnp.float32(NSEL))
    mask = rm >= t
    mf = mask.astype(F32)
    cs = jnp.dot(mf, _tri_incl(128), precision=HI,
                 preferred_element_type=F32)
    rowtot = cs[:, -1:]
    off = jnp.dot(_tril(RPAD), rowtot, precision=HI,
                  preferred_element_type=F32)
    inclr = off + rowtot
    c_tot = jnp.sum(mf)
    identr = (jax.lax.broadcasted_iota(jnp.int32, (RPAD, RPAD), 0) ==
              jax.lax.broadcasted_iota(jnp.int32, (RPAD, RPAD), 1)).astype(F32)
    inclr_row = _col_to_row(inclr, identr)
    off_row = _col_to_row(off, identr)
    pcol = _iotaf((CAP, 1), 0)
    rp = jnp.sum((inclr_row <= pcol).astype(F32), axis=1, keepdims=True)
    rio = _iotaf((1, RPAD), 1)
    oh2 = (rp == rio).astype(F32)
    startp = jnp.sum(oh2 * off_row, axis=1, keepdims=True)
    q = pcol - startp
    csm = jnp.where(mask, cs - mf, -1.0)
    csg = jnp.dot(oh2, csm, precision=HI, preferred_element_type=F32)
    ohc = csg == q
    laneio = _iotaf((CAP, 128), 1)
    lanesel = jnp.sum(jnp.where(ohc, laneio, 0.0), axis=1, keepdims=True)
    rowid = rp * 128.0 + lanesel
    rows_ref[0] = jnp.where(pcol < c_tot, rowid + 1.0, 0.0)


def _run_k2(rm):
    return pl.pallas_call(
        _k2,
        grid=(B,),
        compiler_params=pltpu.CompilerParams(
            dimension_semantics=("parallel",)),
        in_specs=[pl.BlockSpec((1, RPAD, 128), lambda b: (b, 0, 0))],
        out_specs=pl.BlockSpec((1, CAP, 1), lambda b: (b, 0, 0)),
        out_shape=jax.ShapeDtypeStruct((B, CAP, 1), F32),
    )(rm)


def _k3(logits_ref, wt_ref, rowsf_ref, rowsi_ref, sc_ref, lab_ref, br_ref,
        lrows, sem):
    b = pl.program_id(0)

    def start_body(i, carry):
        r = rowsi_ref[b, i]
        pltpu.make_async_copy(logits_ref.at[b, pl.ds(r, 1), :],
                              lrows.at[pl.ds(i, 1), :], sem).start()
        return carry

    jax.lax.fori_loop(0, CAP, start_body, 0)

    def wait_body(i, carry):
        r = rowsi_ref[b, i]
        pltpu.make_async_copy(logits_ref.at[b, pl.ds(r, 1), :],
                              lrows.at[pl.ds(i, 1), :], sem).wait()
        return carry

    jax.lax.fori_loop(0, CAP, wait_body, 0)
    cand = jnp.dot(jax.nn.sigmoid(lrows[...]), wt_ref[...],
                   preferred_element_type=F32)
    rowsf = rowsf_ref[0]
    valid = rowsf >= 0.0
    V = jnp.where(valid, cand, -1.0)
    te = _refine_threshold(V, jnp.float32(NSEL))
    me = V >= te
    mf = me.astype(F32)
    cs = jnp.dot(mf, _tri_incl(L), precision=HI,
                 preferred_element_type=F32)
    rowtot = cs[:, -1:]
    off = jnp.dot(_tril(CAP), rowtot, precision=HI,
                  preferred_element_type=F32)
    inclr = off + rowtot
    ident = (jax.lax.broadcasted_iota(jnp.int32, (CAP, CAP), 0) ==
             jax.lax.broadcasted_iota(jnp.int32, (CAP, CAP), 1)).astype(F32)
    inclr_row = _col_to_row(inclr, ident)
    off_row = _col_to_row(off, ident)
    rowsf_row = _col_to_row(rowsf, ident)
    pcol = _iotaf((CAP, 1), 0)
    rp = jnp.sum((inclr_row <= pcol).astype(F32), axis=1, keepdims=True)
    rio = _iotaf((1, CAP), 1)
    oh2 = (rp == rio).astype(F32)
    startp = jnp.sum(oh2 * off_row, axis=1, keepdims=True)
    q = pcol - startp
    csm = jnp.where(me, cs - mf, -1.0)
    csg = jnp.dot(oh2, csm, precision=HI, preferred_element_type=F32)
    vg = jnp.dot(oh2, V, precision=HI, preferred_element_type=F32)
    ohc = csg == q
    colio = _iotaf((CAP, L), 1)
    sv = jnp.sum(jnp.where(ohc, vg, 0.0), axis=1, keepdims=True)
    scol = jnp.sum(jnp.where(ohc, colio, 0.0), axis=1, keepdims=True)
    srow = jnp.sum(oh2 * rowsf_row, axis=1, keepdims=True)
    ce = jnp.sum(mf)
    sl = _iotaf((CAP, 1), 0) < ce
    vv = jnp.where(sl, sv, -1.0)
    vflat = jnp.where(sl, srow * jnp.float32(L) + scol, 4.0e6)
    ident = (jax.lax.broadcasted_iota(jnp.int32, (CAP, CAP), 0) ==
             jax.lax.broadcasted_iota(jnp.int32, (CAP, CAP), 1)).astype(F32)
    vvr = _col_to_row(vv, ident)
    vfr = _col_to_row(vflat, ident)
    gt = (vvr > vv) | ((vvr == vv) & (vfr < vflat))
    rank = jnp.sum(gt.astype(F32), axis=1, keepdims=True)
    rio = _iotaf((1, CAP), 1)
    oh_sel = (rank == rio).astype(F32)
    sc_ref[0] = jnp.sum(oh_sel * vv, axis=0, keepdims=True)
    lab_ref[0] = jnp.sum(oh_sel * scol, axis=0, keepdims=True)
    br_ref[0] = jnp.sum(oh_sel * srow, axis=0, keepdims=True)


def _run_k3(pred_logits, wt, rowsf, rowsi):
    out = pl.BlockSpec((1, 1, CAP), lambda b: (b, 0, 0))
    return pl.pallas_call(
        _k3,
        grid=(B,),
        compiler_params=pltpu.CompilerParams(
            dimension_semantics=("parallel",)),
        in_specs=[
            pl.BlockSpec(memory_space=pl.ANY),
            pl.BlockSpec((D, L), lambda b: (0, 0)),
            pl.BlockSpec((1, CAP, 1), lambda b: (b, 0, 0)),
            pl.BlockSpec(memory_space=pltpu.SMEM),
        ],
        out_specs=[out, out, out],
        out_shape=[jax.ShapeDtypeStruct((B, 1, CAP), F32)] * 3,
        scratch_shapes=[pltpu.VMEM((CAP, D), F32),
                        pltpu.SemaphoreType.DMA],
    )(pred_logits, wt, rowsf, rowsi)


def _k5(browi_ref, pb_ref, tsz_ref, out_ref):
    b = pl.program_id(0)

    def body(i, carry):
        r = browi_ref[b, i]
        out_ref[0, pl.ds(i, 1), :] = pb_ref[0, pl.ds(r, 1), :]
        return carry

    jax.lax.fori_loop(0, CAP, body, 0)
    g = out_ref[0]
    cx, cy, w, h = g[:, 0:1], g[:, 1:2], g[:, 2:3], g[:, 3:4]
    x0 = cx - 0.5 * w
    y0 = cy - 0.5 * h
    x1 = cx + 0.5 * w
    y1 = cy + 0.5 * h
    wf = tsz_ref[b, 1].astype(F32)
    hf = tsz_ref[b, 0].astype(F32)
    out_ref[0] = jnp.concatenate([x0 * wf, y0 * hf, x1 * wf, y1 * hf], axis=1)


def _run_k5(browi, pred_boxes, target_sizes):
    return pl.pallas_call(
        _k5,
        grid=(B,),
        compiler_params=pltpu.CompilerParams(
            dimension_semantics=("parallel",)),
        in_specs=[
            pl.BlockSpec(memory_space=pltpu.SMEM),
            pl.BlockSpec((1, N, 4), lambda b: (b, 0, 0)),
            pl.BlockSpec(memory_space=pltpu.SMEM),
        ],
        out_specs=pl.BlockSpec((1, CAP, 4), lambda b: (b, 0, 0)),
        out_shape=jax.ShapeDtypeStruct((B, CAP, 4), F32),
    )(browi, pred_boxes, target_sizes)


@jax.jit
def kernel(pred_logits, pred_boxes, target_sizes, positive_maps):
    wt = jnp.swapaxes(positive_maps, 1, 2)[0]
    rowmax4 = _run_k1(pred_logits, wt)
    rowmax = rowmax4.reshape(B, N)
    rm = jnp.pad(rowmax, ((0, 0), (0, RPAD * 128 - N)),
                 constant_values=-1.0).reshape(B, RPAD, 128)
    rows_raw = _run_k2(rm)
    rowsf = rows_raw - 1.0
    rowsi = jnp.maximum(rows_raw[:, :, 0].astype(jnp.int32) - 1, 0)
    scores, labf, browf = _run_k3(pred_logits, wt, rowsf, rowsi)
    browi = jnp.clip(browf[:, 0, :].astype(jnp.int32), 0, N - 1)
    boxes = _run_k5(browi, pred_boxes, target_sizes)
    return (scores[:, 0, :NSEL],
            labf[:, 0, :NSEL].astype(jnp.int32),
            boxes[:, :NSEL, :])
